# speedup vs baseline: 1.7357x; 1.7357x over previous
import functools

import jax
import jax.numpy as jnp
import numpy as np
from jax.experimental import pallas as pl
from jax.experimental.pallas import tpu as pltpu

XMIN = 0.1
XMAX = 2.0

_HI = jax.lax.Precision.HIGHEST


def _fused_kernel(flat_ref, s2_ref, ph_ref, lo_ref, hi_ref, w_ref, b_ref,
                  out_ref, ytop_ref, segacc_ref, *, T, K, E, B):
    i = pl.program_id(0)
    k = jax.lax.rem(i, K)
    base = k * T
    idx = jax.lax.broadcasted_iota(jnp.int32, (T, B), 0) + base
    lo = lo_ref[...]
    hi = hi_ref[...]
    onehot = jnp.where((idx >= lo) & (idx < hi), 1.0, 0.0)

    @pl.when(i < K)
    def _phase_a():
        @pl.when(i == 0)
        def _init():
            segacc_ref[...] = jnp.zeros_like(segacc_ref)

        x = flat_ref[...]
        ang = jax.lax.dot_general(x, s2_ref[...], (((1,), (0,)), ((), ())),
                                  preferred_element_type=jnp.float32,
                                  precision=_HI)
        emb = jnp.sin(ang + ph_ref[...])
        y_top = jax.lax.dot_general(emb, w_ref[:E, :], (((1,), (0,)), ((), ())),
                                    preferred_element_type=jnp.float32,
                                    precision=_HI)
        y_bot = jax.lax.dot_general(emb, w_ref[E:, :], (((1,), (0,)), ((), ())),
                                    preferred_element_type=jnp.float32,
                                    precision=_HI)
        ytop_ref[pl.ds(base, T), :] = y_top + b_ref[...]
        segacc_ref[...] += jax.lax.dot_general(
            onehot, y_bot, (((0,), (0,)), ((), ())),
            preferred_element_type=jnp.float32, precision=_HI)

    @pl.when(i >= K)
    def _phase_b():
        inv_cnt = 1.0 / jnp.maximum((hi - lo).astype(jnp.float32), 1.0)
        scaled = onehot * inv_cnt
        ctx = jax.lax.dot_general(scaled, segacc_ref[...],
                                  (((1,), (0,)), ((), ())),
                                  preferred_element_type=jnp.float32,
                                  precision=_HI)
        out_ref[...] = ytop_ref[pl.ds(base, T), :] + ctx


def kernel(flat, cu_seqlens, W, b):
    n, d = flat.shape
    B = cu_seqlens.shape[0] - 1
    out_dim = W.shape[1]
    enc = W.shape[0] // (2 * d)
    half = enc // 2
    E = d * enc

    scales = XMIN * (XMAX / XMIN) ** (np.arange(half, dtype=np.float64)
                                      / max(half - 1, 1))
    phase = np.zeros((E,), np.float32)
    s2 = np.zeros((d, E), np.float32)
    for c in range(E):
        j = c % enc
        phase[c] = 0.0 if j < half else np.pi / 2.0
        s2[c // enc, c] = 1.0 / scales[j % half]
    s2 = jnp.asarray(s2)
    phase = jnp.asarray(phase).reshape(1, E)

    lo = cu_seqlens[:-1].reshape(1, B).astype(jnp.int32)
    hi = cu_seqlens[1:].reshape(1, B).astype(jnp.int32)
    b2 = b.reshape(1, out_dim)

    T = 2048
    K = n // T

    body = functools.partial(_fused_kernel, T=T, K=K, E=E, B=B)
    out = pl.pallas_call(
        body,
        grid=(2 * K,),
        in_specs=[
            pl.BlockSpec((T, d), lambda i: (i % K, 0)),
            pl.BlockSpec((d, E), lambda i: (0, 0)),
            pl.BlockSpec((1, E), lambda i: (0, 0)),
            pl.BlockSpec((1, B), lambda i: (0, 0)),
            pl.BlockSpec((1, B), lambda i: (0, 0)),
            pl.BlockSpec((2 * E, out_dim), lambda i: (0, 0)),
            pl.BlockSpec((1, out_dim), lambda i: (0, 0)),
        ],
        out_specs=pl.BlockSpec((T, out_dim), lambda i: (i % K, 0)),
        out_shape=jax.ShapeDtypeStruct((n, out_dim), jnp.float32),
        scratch_shapes=[
            pltpu.VMEM((n, out_dim), jnp.float32),
            pltpu.VMEM((B, out_dim), jnp.float32),
        ],
    )(flat, s2, phase, lo, hi, W, b2)
    return out


# speedup vs baseline: 2.2965x; 1.3231x over previous
import functools

import jax
import jax.numpy as jnp
import numpy as np
from jax.experimental import pallas as pl
from jax.experimental.pallas import tpu as pltpu

XMIN = 0.1
XMAX = 2.0

_HI = jax.lax.Precision.HIGHEST


def _fused_kernel(flat_ref, s2_ref, ph_ref, lo_ref, hi_ref, w_ref, b_ref,
                  out_ref, ytop_ref, segacc_ref, *, T, K, E, B):
    i = pl.program_id(0)
    k = jax.lax.rem(i, K)
    base = k * T
    idx = jax.lax.broadcasted_iota(jnp.int32, (T, B), 0) + base
    lo = lo_ref[...]
    hi = hi_ref[...]
    onehot = jnp.where((idx >= lo) & (idx < hi), 1.0, 0.0)

    @pl.when(i < K)
    def _phase_a():
        @pl.when(i == 0)
        def _init():
            segacc_ref[...] = jnp.zeros_like(segacc_ref)

        x = flat_ref[...]
        ang = jax.lax.dot_general(x, s2_ref[...], (((1,), (0,)), ((), ())),
                                  preferred_element_type=jnp.float32,
                                  precision=_HI)
        emb = jnp.sin(ang + ph_ref[...])
        OUT = out_ref.shape[-1]
        y = jax.lax.dot_general(emb, w_ref[...], (((1,), (0,)), ((), ())),
                                preferred_element_type=jnp.float32)
        ytop_ref[pl.ds(base, T), :] = y[:, :OUT] + b_ref[...]
        segacc_ref[...] += jax.lax.dot_general(
            onehot, y[:, OUT:], (((0,), (0,)), ((), ())),
            preferred_element_type=jnp.float32)

    @pl.when(i >= K)
    def _phase_b():
        inv_cnt = 1.0 / jnp.maximum((hi - lo).astype(jnp.float32), 1.0)
        scaled = onehot * inv_cnt
        ctx = jax.lax.dot_general(scaled, segacc_ref[...],
                                  (((1,), (0,)), ((), ())),
                                  preferred_element_type=jnp.float32)
        out_ref[...] = ytop_ref[pl.ds(base, T), :] + ctx


def kernel(flat, cu_seqlens, W, b):
    n, d = flat.shape
    B = cu_seqlens.shape[0] - 1
    out_dim = W.shape[1]
    enc = W.shape[0] // (2 * d)
    half = enc // 2
    E = d * enc

    scales = XMIN * (XMAX / XMIN) ** (np.arange(half, dtype=np.float64)
                                      / max(half - 1, 1))
    phase = np.zeros((E,), np.float32)
    s2 = np.zeros((d, E), np.float32)
    for c in range(E):
        j = c % enc
        phase[c] = 0.0 if j < half else np.pi / 2.0
        s2[c // enc, c] = 1.0 / scales[j % half]
    s2 = jnp.asarray(s2)
    phase = jnp.asarray(phase).reshape(1, E)

    lo = cu_seqlens[:-1].reshape(1, B).astype(jnp.int32)
    hi = cu_seqlens[1:].reshape(1, B).astype(jnp.int32)
    b2 = b.reshape(1, out_dim)
    w2 = jnp.concatenate([W[:E, :], W[E:, :]], axis=1)

    T = 2048
    K = n // T

    body = functools.partial(_fused_kernel, T=T, K=K, E=E, B=B)
    out = pl.pallas_call(
        body,
        grid=(2 * K,),
        in_specs=[
            pl.BlockSpec((T, d), lambda i: (i % K, 0)),
            pl.BlockSpec((d, E), lambda i: (0, 0)),
            pl.BlockSpec((1, E), lambda i: (0, 0)),
            pl.BlockSpec((1, B), lambda i: (0, 0)),
            pl.BlockSpec((1, B), lambda i: (0, 0)),
            pl.BlockSpec((E, 2 * out_dim), lambda i: (0, 0)),
            pl.BlockSpec((1, out_dim), lambda i: (0, 0)),
        ],
        out_specs=pl.BlockSpec((T, out_dim), lambda i: (i % K, 0)),
        out_shape=jax.ShapeDtypeStruct((n, out_dim), jnp.float32),
        scratch_shapes=[
            pltpu.VMEM((n, out_dim), jnp.float32),
            pltpu.VMEM((B, out_dim), jnp.float32),
        ],
    )(flat, s2, phase, lo, hi, w2, b2)
    return out


# speedup vs baseline: 3.6224x; 1.5773x over previous
import functools

import jax
import jax.numpy as jnp
import numpy as np
from jax.experimental import pallas as pl
from jax.experimental.pallas import tpu as pltpu

XMIN = 0.1
XMAX = 2.0

_HI = jax.lax.Precision.HIGHEST

_PIO2_H1 = np.float32(1.5703125)
_PIO2_H2 = np.float32(np.pi / 2 - 1.5703125)
_PIO2_H3 = np.float32(np.pi / 2 - 1.5703125 - float(np.float32(np.pi / 2 - 1.5703125)))
_INV_PIO2 = np.float32(2.0 / np.pi)
_S1 = np.float32(-1.6666654611e-1)
_S2 = np.float32(8.3321608736e-3)
_S3 = np.float32(-1.9515295891e-4)
_C1 = np.float32(4.166664568298827e-2)
_C2 = np.float32(-1.388731625493765e-3)
_C3 = np.float32(2.443315711809948e-5)


def _fast_sin(ang):
    nf = jnp.floor(ang * _INV_PIO2 + 0.5)
    r = ang - nf * _PIO2_H1
    r = r - nf * _PIO2_H2
    r = r - nf * _PIO2_H3
    ni = nf.astype(jnp.int32)
    r2 = r * r
    sp = ((_S3 * r2 + _S2) * r2 + _S1) * (r2 * r) + r
    cp = ((_C3 * r2 + _C2) * r2 + _C1) * (r2 * r2) + (1.0 - 0.5 * r2)
    res = jnp.where((ni & 1) == 0, sp, cp)
    return jnp.where((ni & 2) == 0, res, -res)


def _fused_kernel(flat_ref, s2_ref, ph_ref, lo_ref, hi_ref, w_ref, b_ref,
                  out_ref, ytop_ref, segacc_ref, *, T, K, E, B):
    i = pl.program_id(0)
    k = jax.lax.rem(i, K)
    base = k * T
    idx = jax.lax.broadcasted_iota(jnp.int32, (T, B), 0) + base
    lo = lo_ref[...]
    hi = hi_ref[...]
    onehot = jnp.where((idx >= lo) & (idx < hi), 1.0, 0.0)

    @pl.when(i < K)
    def _phase_a():
        @pl.when(i == 0)
        def _init():
            segacc_ref[...] = jnp.zeros_like(segacc_ref)

        x = flat_ref[...]
        ang = jax.lax.dot_general(x, s2_ref[...], (((1,), (0,)), ((), ())),
                                  preferred_element_type=jnp.float32,
                                  precision=_HI)
        emb = _fast_sin(ang + ph_ref[...])
        OUT = out_ref.shape[-1]
        y = jax.lax.dot_general(emb, w_ref[...], (((1,), (0,)), ((), ())),
                                preferred_element_type=jnp.float32)
        ytop_ref[pl.ds(base, T), :] = y[:, :OUT] + b_ref[...]
        segacc_ref[...] += jax.lax.dot_general(
            onehot, y[:, OUT:], (((0,), (0,)), ((), ())),
            preferred_element_type=jnp.float32)

    @pl.when(i >= K)
    def _phase_b():
        inv_cnt = 1.0 / jnp.maximum((hi - lo).astype(jnp.float32), 1.0)
        scaled = onehot * inv_cnt
        ctx = jax.lax.dot_general(scaled, segacc_ref[...],
                                  (((1,), (0,)), ((), ())),
                                  preferred_element_type=jnp.float32)
        out_ref[...] = ytop_ref[pl.ds(base, T), :] + ctx


def kernel(flat, cu_seqlens, W, b):
    n, d = flat.shape
    B = cu_seqlens.shape[0] - 1
    out_dim = W.shape[1]
    enc = W.shape[0] // (2 * d)
    half = enc // 2
    E = d * enc

    scales = XMIN * (XMAX / XMIN) ** (np.arange(half, dtype=np.float64)
                                      / max(half - 1, 1))
    phase = np.zeros((E,), np.float32)
    s2 = np.zeros((d, E), np.float32)
    for c in range(E):
        j = c % enc
        phase[c] = 0.0 if j < half else np.pi / 2.0
        s2[c // enc, c] = 1.0 / scales[j % half]
    s2 = jnp.asarray(s2)
    phase = jnp.asarray(phase).reshape(1, E)

    lo = cu_seqlens[:-1].reshape(1, B).astype(jnp.int32)
    hi = cu_seqlens[1:].reshape(1, B).astype(jnp.int32)
    b2 = b.reshape(1, out_dim)
    w2 = jnp.concatenate([W[:E, :], W[E:, :]], axis=1)

    T = 2048
    K = n // T

    body = functools.partial(_fused_kernel, T=T, K=K, E=E, B=B)
    out = pl.pallas_call(
        body,
        grid=(2 * K,),
        in_specs=[
            pl.BlockSpec((T, d), lambda i: (i % K, 0)),
            pl.BlockSpec((d, E), lambda i: (0, 0)),
            pl.BlockSpec((1, E), lambda i: (0, 0)),
            pl.BlockSpec((1, B), lambda i: (0, 0)),
            pl.BlockSpec((1, B), lambda i: (0, 0)),
            pl.BlockSpec((E, 2 * out_dim), lambda i: (0, 0)),
            pl.BlockSpec((1, out_dim), lambda i: (0, 0)),
        ],
        out_specs=pl.BlockSpec((T, out_dim), lambda i: (i % K, 0)),
        out_shape=jax.ShapeDtypeStruct((n, out_dim), jnp.float32),
        scratch_shapes=[
            pltpu.VMEM((n, out_dim), jnp.float32),
            pltpu.VMEM((B, out_dim), jnp.float32),
        ],
    )(flat, s2, phase, lo, hi, w2, b2)
    return out


# speedup vs baseline: 3.7031x; 1.0223x over previous
import functools

import jax
import jax.numpy as jnp
import numpy as np
from jax.experimental import pallas as pl
from jax.experimental.pallas import tpu as pltpu

XMIN = 0.1
XMAX = 2.0

_HI = jax.lax.Precision.HIGHEST

_PIO2_H1 = np.float32(1.5703125)
_PIO2_H2 = np.float32(np.pi / 2 - 1.5703125)
_PIO2_H3 = np.float32(np.pi / 2 - 1.5703125 - float(np.float32(np.pi / 2 - 1.5703125)))
_INV_PIO2 = np.float32(2.0 / np.pi)
_S1 = np.float32(-1.6666654611e-1)
_S2 = np.float32(8.3321608736e-3)
_S3 = np.float32(-1.9515295891e-4)
_C1 = np.float32(4.166664568298827e-2)
_C2 = np.float32(-1.388731625493765e-3)
_C3 = np.float32(2.443315711809948e-5)


def _fast_sin(ang):
    nf = jnp.floor(ang * _INV_PIO2 + 0.5)
    r = ang - nf * _PIO2_H1
    r = r - nf * _PIO2_H2
    r = r - nf * _PIO2_H3
    ni = nf.astype(jnp.int32)
    r2 = r * r
    sp = ((_S3 * r2 + _S2) * r2 + _S1) * (r2 * r) + r
    cp = ((_C3 * r2 + _C2) * r2 + _C1) * (r2 * r2) + (1.0 - 0.5 * r2)
    res = jnp.where((ni & 1) == 0, sp, cp)
    return jnp.where((ni & 2) == 0, res, -res)


def _onehot(base, lo_ref, hi_ref, T, B):
    idx = jax.lax.broadcasted_iota(jnp.int32, (T, B), 0) + base
    return jnp.where((idx >= lo_ref[...]) & (idx < hi_ref[...]), 1.0, 0.0)


def _pass_a(flat_ref, s2_ref, ph_ref, lo_ref, hi_ref, w_ref, b_ref,
            ytop_ref, segacc_ref, *, T, E, B):
    i = pl.program_id(0)
    onehot = _onehot(i * T, lo_ref, hi_ref, T, B)
    x = flat_ref[...]
    ang = jax.lax.dot_general(x, s2_ref[...], (((1,), (0,)), ((), ())),
                              preferred_element_type=jnp.float32,
                              precision=_HI)
    emb = _fast_sin(ang + ph_ref[...])
    OUT = ytop_ref.shape[-1]
    y = jax.lax.dot_general(emb, w_ref[...], (((1,), (0,)), ((), ())),
                            preferred_element_type=jnp.float32)
    ytop_ref[...] = y[:, :OUT] + b_ref[...]
    part = jax.lax.dot_general(onehot, y[:, OUT:], (((0,), (0,)), ((), ())),
                               preferred_element_type=jnp.float32)

    @pl.when(i == 0)
    def _init():
        segacc_ref[...] = part

    @pl.when(i != 0)
    def _acc():
        segacc_ref[...] += part


def _pass_b(ytop_ref, lo_ref, hi_ref, segacc_ref, out_ref, *, T, B):
    i = pl.program_id(0)
    lo = lo_ref[...]
    hi = hi_ref[...]
    onehot = _onehot(i * T, lo_ref, hi_ref, T, B)
    inv_cnt = 1.0 / jnp.maximum((hi - lo).astype(jnp.float32), 1.0)
    ctx = jax.lax.dot_general(onehot * inv_cnt, segacc_ref[...],
                              (((1,), (0,)), ((), ())),
                              preferred_element_type=jnp.float32)
    out_ref[...] = ytop_ref[...] + ctx


def kernel(flat, cu_seqlens, W, b):
    n, d = flat.shape
    B = cu_seqlens.shape[0] - 1
    out_dim = W.shape[1]
    enc = W.shape[0] // (2 * d)
    half = enc // 2
    E = d * enc

    scales = XMIN * (XMAX / XMIN) ** (np.arange(half, dtype=np.float64)
                                      / max(half - 1, 1))
    phase = np.zeros((E,), np.float32)
    s2 = np.zeros((d, E), np.float32)
    for c in range(E):
        j = c % enc
        phase[c] = 0.0 if j < half else np.pi / 2.0
        s2[c // enc, c] = 1.0 / scales[j % half]
    s2 = jnp.asarray(s2)
    phase = jnp.asarray(phase).reshape(1, E)

    lo = cu_seqlens[:-1].reshape(1, B).astype(jnp.int32)
    hi = cu_seqlens[1:].reshape(1, B).astype(jnp.int32)
    b2 = b.reshape(1, out_dim)
    w2 = jnp.concatenate([W[:E, :], W[E:, :]], axis=1)

    T = 2048
    K = n // T

    ytop, segacc = pl.pallas_call(
        functools.partial(_pass_a, T=T, E=E, B=B),
        grid=(K,),
        in_specs=[
            pl.BlockSpec((T, d), lambda i: (i, 0)),
            pl.BlockSpec((d, E), lambda i: (0, 0)),
            pl.BlockSpec((1, E), lambda i: (0, 0)),
            pl.BlockSpec((1, B), lambda i: (0, 0)),
            pl.BlockSpec((1, B), lambda i: (0, 0)),
            pl.BlockSpec((E, 2 * out_dim), lambda i: (0, 0)),
            pl.BlockSpec((1, out_dim), lambda i: (0, 0)),
        ],
        out_specs=[
            pl.BlockSpec((T, out_dim), lambda i: (i, 0)),
            pl.BlockSpec((B, out_dim), lambda i: (0, 0)),
        ],
        out_shape=[
            jax.ShapeDtypeStruct((n, out_dim), jnp.float32),
            jax.ShapeDtypeStruct((B, out_dim), jnp.float32),
        ],
    )(flat, s2, phase, lo, hi, w2, b2)

    out = pl.pallas_call(
        functools.partial(_pass_b, T=T, B=B),
        grid=(K,),
        in_specs=[
            pl.BlockSpec((T, out_dim), lambda i: (i, 0)),
            pl.BlockSpec((1, B), lambda i: (0, 0)),
            pl.BlockSpec((1, B), lambda i: (0, 0)),
            pl.BlockSpec((B, out_dim), lambda i: (0, 0)),
        ],
        out_specs=pl.BlockSpec((T, out_dim), lambda i: (i, 0)),
        out_shape=jax.ShapeDtypeStruct((n, out_dim), jnp.float32),
    )(ytop, lo, hi, segacc)
    return out


# speedup vs baseline: 5.4404x; 1.4691x over previous
import functools

import jax
import jax.numpy as jnp
import numpy as np
from jax.experimental import pallas as pl
from jax.experimental.pallas import tpu as pltpu

XMIN = 0.1
XMAX = 2.0

_HI = jax.lax.Precision.HIGHEST

_PIO2_H1 = np.float32(1.5703125)
_PIO2_H2 = np.float32(np.pi / 2 - 1.5703125)
_PIO2_H3 = np.float32(np.pi / 2 - 1.5703125 - float(np.float32(np.pi / 2 - 1.5703125)))
_INV_PIO2 = np.float32(2.0 / np.pi)
_S1 = np.float32(-1.6666654611e-1)
_S2 = np.float32(8.3321608736e-3)
_S3 = np.float32(-1.9515295891e-4)
_C1 = np.float32(4.166664568298827e-2)
_C2 = np.float32(-1.388731625493765e-3)
_C3 = np.float32(2.443315711809948e-5)


def _fast_sin(ang):
    nf = jnp.floor(ang * _INV_PIO2 + 0.5)
    r = ang - nf * _PIO2_H1
    r = r - nf * _PIO2_H2
    r = r - nf * _PIO2_H3
    ni = nf.astype(jnp.int32)
    r2 = r * r
    sp = ((_S3 * r2 + _S2) * r2 + _S1) * (r2 * r) + r
    cp = ((_C3 * r2 + _C2) * r2 + _C1) * (r2 * r2) + (1.0 - 0.5 * r2)
    res = jnp.where((ni & 1) == 0, sp, cp)
    return jnp.where((ni & 2) == 0, res, -res)


def _onehot(base, lo_ref, hi_ref, T, B):
    idx = jax.lax.broadcasted_iota(jnp.int32, (T, B), 0) + base
    return jnp.where((idx >= lo_ref[...]) & (idx < hi_ref[...]), 1.0, 0.0)


def _pass_a(flat_ref, s2_ref, lo_ref, hi_ref, w_ref, b_ref,
            ytop_ref, segacc_ref, *, T, E, B):
    i = pl.program_id(0)
    onehot = _onehot(i * T, lo_ref, hi_ref, T, B)
    x = flat_ref[...]
    ang = jax.lax.dot_general(x, s2_ref[...], (((1,), (0,)), ((), ())),
                              preferred_element_type=jnp.float32)
    nf = jnp.floor(ang * _INV_PIO2 + 0.5)
    r = ang - nf * _PIO2_H1
    r = r - nf * _PIO2_H2
    r = r - nf * _PIO2_H3
    ni = nf.astype(jnp.int32)
    r2 = r * r
    sp = ((_S3 * r2 + _S2) * r2 + _S1) * (r2 * r) + r
    cp = ((_C3 * r2 + _C2) * r2 + _C1) * (r2 * r2) + (1.0 - 0.5 * r2)
    odd = (ni & 1) == 0
    sinv = jnp.where(odd, sp, cp)
    sinv = jnp.where((ni & 2) == 0, sinv, -sinv)
    cosv = jnp.where(odd, cp, sp)
    cosv = jnp.where(((ni + 1) & 2) == 0, cosv, -cosv)
    emb = jnp.concatenate([sinv, cosv], axis=1)
    OUT = ytop_ref.shape[-1]
    y = jax.lax.dot_general(emb, w_ref[...], (((1,), (0,)), ((), ())),
                            preferred_element_type=jnp.float32)
    ytop_ref[...] = y[:, :OUT] + b_ref[...]
    part = jax.lax.dot_general(onehot, y[:, OUT:], (((0,), (0,)), ((), ())),
                               preferred_element_type=jnp.float32)

    @pl.when(i == 0)
    def _init():
        segacc_ref[...] = part

    @pl.when(i != 0)
    def _acc():
        segacc_ref[...] += part


def _pass_b(ytop_ref, lo_ref, hi_ref, segacc_ref, out_ref, *, T, B):
    i = pl.program_id(0)
    lo = lo_ref[...]
    hi = hi_ref[...]
    onehot = _onehot(i * T, lo_ref, hi_ref, T, B)
    inv_cnt = 1.0 / jnp.maximum((hi - lo).astype(jnp.float32), 1.0)
    ctx = jax.lax.dot_general(onehot * inv_cnt, segacc_ref[...],
                              (((1,), (0,)), ((), ())),
                              preferred_element_type=jnp.float32)
    out_ref[...] = ytop_ref[...] + ctx


def kernel(flat, cu_seqlens, W, b):
    n, d = flat.shape
    B = cu_seqlens.shape[0] - 1
    out_dim = W.shape[1]
    enc = W.shape[0] // (2 * d)
    half = enc // 2
    E = d * enc

    scales = XMIN * (XMAX / XMIN) ** (np.arange(half, dtype=np.float64)
                                      / max(half - 1, 1))
    Eh = E // 2
    s2 = np.zeros((d, Eh), np.float32)
    for f in range(d):
        for j in range(half):
            s2[f, f * half + j] = 1.0 / scales[j]
    s2 = jnp.asarray(s2)

    lo = cu_seqlens[:-1].reshape(1, B).astype(jnp.int32)
    hi = cu_seqlens[1:].reshape(1, B).astype(jnp.int32)
    b2 = b.reshape(1, out_dim)
    w2 = jnp.concatenate([W[:E, :], W[E:, :]], axis=1)
    sin_rows = np.array([f * enc + j for f in range(d) for j in range(half)])
    perm = np.concatenate([sin_rows, sin_rows + half])
    w2 = w2[perm, :]

    T = 2048
    K = n // T

    ytop, segacc = pl.pallas_call(
        functools.partial(_pass_a, T=T, E=E, B=B),
        grid=(K,),
        in_specs=[
            pl.BlockSpec((T, d), lambda i: (i, 0)),
            pl.BlockSpec((d, E // 2), lambda i: (0, 0)),
            pl.BlockSpec((1, B), lambda i: (0, 0)),
            pl.BlockSpec((1, B), lambda i: (0, 0)),
            pl.BlockSpec((E, 2 * out_dim), lambda i: (0, 0)),
            pl.BlockSpec((1, out_dim), lambda i: (0, 0)),
        ],
        out_specs=[
            pl.BlockSpec((T, out_dim), lambda i: (i, 0)),
            pl.BlockSpec((B, out_dim), lambda i: (0, 0)),
        ],
        out_shape=[
            jax.ShapeDtypeStruct((n, out_dim), jnp.float32),
            jax.ShapeDtypeStruct((B, out_dim), jnp.float32),
        ],
    )(flat, s2, lo, hi, w2, b2)

    out = pl.pallas_call(
        functools.partial(_pass_b, T=T, B=B),
        grid=(K,),
        in_specs=[
            pl.BlockSpec((T, out_dim), lambda i: (i, 0)),
            pl.BlockSpec((1, B), lambda i: (0, 0)),
            pl.BlockSpec((1, B), lambda i: (0, 0)),
            pl.BlockSpec((B, out_dim), lambda i: (0, 0)),
        ],
        out_specs=pl.BlockSpec((T, out_dim), lambda i: (i, 0)),
        out_shape=jax.ShapeDtypeStruct((n, out_dim), jnp.float32),
    )(ytop, lo, hi, segacc)
    return out
